# baseline (speedup 1.0000x reference)
import functools

import jax
import jax.numpy as jnp
from jax.experimental import pallas as pl
from jax.experimental.pallas import tpu as pltpu


B, S, D = 16, 4096, 512
T = 512


def _prep_body(mask_ref, lens_ref, rest_ref):
    lens = jnp.sum(mask_ref[...], axis=1, keepdims=True)
    lens_ref[...] = lens.astype(jnp.int32)
    ii = jax.lax.broadcasted_iota(jnp.int32, (B, B), 0)
    jj = jax.lax.broadcasted_iota(jnp.int32, (B, B), 1)
    diagm = jnp.where(ii == jj, lens, 0.0)
    lens_j = jnp.sum(diagm, axis=0, keepdims=True)
    gt = lens_j > lens
    tie = (lens_j == lens) & (jj < ii)
    rank = jnp.sum((gt | tie).astype(jnp.int32), axis=1, keepdims=True)
    rest_ref[...] = rank


def _main_body(lens_ref, x_ref, out_ref, final_ref):
    b = pl.program_id(0)
    t = pl.program_id(1)
    len_b = lens_ref[b]
    t0 = t * T

    @pl.when(t0 < len_b)
    def _():
        idx = jax.lax.broadcasted_iota(jnp.int32, (T, 1), 0) + t0
        valid = (idx < len_b).astype(x_ref.dtype)
        out_ref[0] = x_ref[0] * valid

    @pl.when(t0 >= len_b)
    def _():
        out_ref[0] = jnp.zeros((T, D), x_ref.dtype)

    last = len_b - 1
    @pl.when((last >= t0) & (last < t0 + T))
    def _():
        final_ref[0, 0, :] = x_ref[0, last - t0, :]


@functools.partial(jax.jit, static_argnames=("interpret",))
def kernel(inputs, mask, interpret=False):
    mask_f = mask.astype(jnp.float32)
    lens, rest = pl.pallas_call(
        _prep_body,
        out_shape=[
            jax.ShapeDtypeStruct((B, 1), jnp.int32),
            jax.ShapeDtypeStruct((B, 1), jnp.int32),
        ],
        interpret=interpret,
    )(mask_f)
    lens1d = lens.reshape(B)

    def x_index(b, t, lens_ref):
        return (b, jnp.minimum(t, (lens_ref[b] - 1) // T), 0)

    outputs, final = pl.pallas_call(
        _main_body,
        grid_spec=pltpu.PrefetchScalarGridSpec(
            num_scalar_prefetch=1,
            grid=(B, S // T),
            in_specs=[pl.BlockSpec((1, T, D), x_index)],
            out_specs=[
                pl.BlockSpec((1, T, D), lambda b, t, lens_ref: (b, t, 0)),
                pl.BlockSpec((1, 1, D), lambda b, t, lens_ref: (b, 0, 0)),
            ],
        ),
        out_shape=[
            jax.ShapeDtypeStruct((B, S, D), inputs.dtype),
            jax.ShapeDtypeStruct((B, 1, D), inputs.dtype),
        ],
        interpret=interpret,
    )(lens1d, inputs)
    return outputs, final.reshape(B, D), rest.reshape(B)


# speedup vs baseline: 1.2412x; 1.2412x over previous
import functools

import jax
import jax.numpy as jnp
from jax.experimental import pallas as pl


B, S, D = 16, 4096, 512
T = 2048


def _body(mask_ref, x_ref, out_ref, final_ref, rest_ref):
    b = pl.program_id(0)
    t = pl.program_id(1)

    row = mask_ref[pl.ds(b, 1), :]
    len_b = jnp.sum(row).astype(jnp.int32)

    idx = jax.lax.broadcasted_iota(jnp.int32, (T, 1), 0) + t * T
    valid = (idx < len_b).astype(x_ref.dtype)
    out_ref[0] = x_ref[0] * valid

    last = len_b - 1
    @pl.when((last >= t * T) & (last < (t + 1) * T))
    def _():
        final_ref[0, 0, :] = x_ref[0, last - t * T, :]

    @pl.when((b == 0) & (t == 0))
    def _():
        lens = jnp.sum(mask_ref[...], axis=1, keepdims=True)
        ii = jax.lax.broadcasted_iota(jnp.int32, (B, B), 0)
        jj = jax.lax.broadcasted_iota(jnp.int32, (B, B), 1)
        diagm = jnp.where(ii == jj, lens, 0.0)
        lens_j = jnp.sum(diagm, axis=0, keepdims=True)
        gt = lens_j > lens
        tie = (lens_j == lens) & (jj < ii)
        rank = jnp.sum((gt | tie).astype(jnp.int32), axis=1, keepdims=True)
        rest_ref[...] = rank


@functools.partial(jax.jit, static_argnames=("interpret",))
def kernel(inputs, mask, interpret=False):
    mask_f = mask.astype(jnp.float32)
    outputs, final, rest = pl.pallas_call(
        _body,
        grid=(B, S // T),
        in_specs=[
            pl.BlockSpec((B, S), lambda b, t: (0, 0)),
            pl.BlockSpec((1, T, D), lambda b, t: (b, t, 0)),
        ],
        out_specs=[
            pl.BlockSpec((1, T, D), lambda b, t: (b, t, 0)),
            pl.BlockSpec((1, 1, D), lambda b, t: (b, 0, 0)),
            pl.BlockSpec((B, 1), lambda b, t: (0, 0)),
        ],
        out_shape=[
            jax.ShapeDtypeStruct((B, S, D), inputs.dtype),
            jax.ShapeDtypeStruct((B, 1, D), inputs.dtype),
            jax.ShapeDtypeStruct((B, 1), jnp.int32),
        ],
        interpret=interpret,
    )(mask_f, inputs)
    return outputs, final.reshape(B, D), rest.reshape(B)


# speedup vs baseline: 1.2711x; 1.0241x over previous
import functools

import jax
import jax.numpy as jnp
from jax.experimental import pallas as pl


B, S, D = 16, 4096, 512
T = 4096


def _body(mask_ref, x_ref, out_ref, final_ref, rest_ref):
    b = pl.program_id(0)
    t = pl.program_id(1)

    row = mask_ref[pl.ds(b, 1), :]
    len_b = jnp.sum(row).astype(jnp.int32)

    idx = jax.lax.broadcasted_iota(jnp.int32, (T, 1), 0) + t * T
    valid = (idx < len_b).astype(x_ref.dtype)
    out_ref[0] = x_ref[0] * valid

    last = len_b - 1
    @pl.when((last >= t * T) & (last < (t + 1) * T))
    def _():
        final_ref[0, 0, :] = x_ref[0, last - t * T, :]

    @pl.when((b == 0) & (t == 0))
    def _():
        lens = jnp.sum(mask_ref[...], axis=1, keepdims=True)
        ii = jax.lax.broadcasted_iota(jnp.int32, (B, B), 0)
        jj = jax.lax.broadcasted_iota(jnp.int32, (B, B), 1)
        diagm = jnp.where(ii == jj, lens, 0.0)
        lens_j = jnp.sum(diagm, axis=0, keepdims=True)
        gt = lens_j > lens
        tie = (lens_j == lens) & (jj < ii)
        rank = jnp.sum((gt | tie).astype(jnp.int32), axis=1, keepdims=True)
        rest_ref[...] = rank


@functools.partial(jax.jit, static_argnames=("interpret",))
def kernel(inputs, mask, interpret=False):
    mask_f = mask.astype(jnp.float32)
    outputs, final, rest = pl.pallas_call(
        _body,
        grid=(B, S // T),
        in_specs=[
            pl.BlockSpec((B, S), lambda b, t: (0, 0)),
            pl.BlockSpec((1, T, D), lambda b, t: (b, t, 0)),
        ],
        out_specs=[
            pl.BlockSpec((1, T, D), lambda b, t: (b, t, 0)),
            pl.BlockSpec((1, 1, D), lambda b, t: (b, 0, 0)),
            pl.BlockSpec((B, 1), lambda b, t: (0, 0)),
        ],
        out_shape=[
            jax.ShapeDtypeStruct((B, S, D), inputs.dtype),
            jax.ShapeDtypeStruct((B, 1, D), inputs.dtype),
            jax.ShapeDtypeStruct((B, 1), jnp.int32),
        ],
        interpret=interpret,
    )(mask_f, inputs)
    return outputs, final.reshape(B, D), rest.reshape(B)
